# Initial kernel scaffold; baseline (speedup 1.0000x reference)
import jax
import jax.numpy as jnp
from jax.experimental import pallas as pl


def kernel(source_index, adj, Sfeatures, W1, b1, W2, b2):
    raise NotImplementedError("write your pallas kernel here")


# speedup vs baseline: 3.7255x; 3.7255x over previous
import functools

import jax
import jax.numpy as jnp
from jax import lax
from jax.experimental import pallas as pl
from jax.experimental.pallas import tpu as pltpu
from jax.experimental.pallas import tpu_sc as plsc

_NC = 2
_NS = 16
_NW = _NC * _NS
_LANES = 128


def _table_body(sfeat_ref, adj_ref, w1t_ref, b1_ref, w2t_ref, b2_ref, out_ref):
    h = jnp.dot(sfeat_ref[...], w1t_ref[...], preferred_element_type=jnp.float32)
    h = jnp.maximum(h + b1_ref[...], 0.0)
    h = h * adj_ref[...]
    y = jnp.dot(h, w2t_ref[...], preferred_element_type=jnp.float32)
    y = jnp.maximum(y + b2_ref[...], 0.0)
    m = jnp.max(y, axis=1, keepdims=True)
    lse = jnp.log(jnp.sum(jnp.exp(y - m), axis=1, keepdims=True)) + m
    ls = y - lse
    out_ref[...] = jnp.concatenate([ls, ls], axis=1)


def _compute_table(Sfeatures, adj, W1, b1, W2, b2):
    n = Sfeatures.shape[0]
    hid = W1.shape[0]
    out_f = W2.shape[0]
    return pl.pallas_call(
        _table_body,
        out_shape=jax.ShapeDtypeStruct((n, _LANES), jnp.float32),
    )(Sfeatures, adj, W1.T, b1.reshape(1, hid), W2.T, b2.reshape(1, out_f))


def _gather_rows(table, idx):
    e = idx.shape[0]
    n = table.shape[0]
    per_w = e // _NW
    ch = 200
    n_ch = per_w // ch
    mesh = plsc.VectorSubcoreMesh(
        core_axis_name="c", subcore_axis_name="s", num_cores=_NC, num_subcores=_NS
    )

    @functools.partial(
        pl.kernel,
        mesh=mesh,
        out_type=jax.ShapeDtypeStruct((e, _LANES), jnp.float32),
        scratch_types=[
            pltpu.VMEM_SHARED((n, _LANES), jnp.float32),
            pltpu.VMEM((ch,), jnp.int32),
            pltpu.VMEM((ch, _LANES), jnp.float32),
            pltpu.SemaphoreType.DMA,
        ],
    )
    def gather_k(table_hbm, idx_hbm, out_hbm, table_sh, idx_v, rows_v, sem):
        cid = lax.axis_index("c")
        sid = lax.axis_index("s")
        wid = sid * _NC + cid

        @pl.when(sid == 0)
        def _():
            pltpu.sync_copy(table_hbm, table_sh)

        plsc.subcore_barrier()

        base = wid * per_w

        def body(i, carry):
            off = base + i * ch
            pltpu.sync_copy(idx_hbm.at[pl.ds(off, ch)], idx_v)
            pltpu.async_copy(table_sh.at[idx_v], rows_v, sem).wait()
            pltpu.sync_copy(rows_v, out_hbm.at[pl.ds(off, ch)])
            return carry

        lax.fori_loop(0, n_ch, body, 0)

    return gather_k(table, idx)


def _trim_body(in_ref, out_ref):
    out_ref[...] = in_ref[:, : out_ref.shape[1]]


def _trim_lanes(x, out_f):
    e = x.shape[0]
    bs = 2000
    return pl.pallas_call(
        _trim_body,
        grid=(e // bs,),
        in_specs=[pl.BlockSpec((bs, _LANES), lambda i: (i, 0))],
        out_specs=pl.BlockSpec((bs, out_f), lambda i: (i, 0)),
        out_shape=jax.ShapeDtypeStruct((e, out_f), jnp.float32),
    )(x)


def kernel(source_index, adj, Sfeatures, W1, b1, W2, b2):
    table = _compute_table(Sfeatures, adj, W1, b1, W2, b2)
    gathered = _gather_rows(table, source_index)
    return _trim_lanes(gathered, W2.shape[0])
